# Initial kernel scaffold; baseline (speedup 1.0000x reference)
import jax
import jax.numpy as jnp
from jax.experimental import pallas as pl


def kernel(x, edge_index, rank_mapping, W1, Wp, bp, bias):
    raise NotImplementedError("write your pallas kernel here")


# speedup vs baseline: 271.9736x; 271.9736x over previous
import functools

import jax
import jax.numpy as jnp
from jax import lax
from jax.experimental import pallas as pl
from jax.experimental.pallas import tpu as pltpu
from jax.experimental.pallas import tpu_sc as plsc

N = 10000
E = 320000
D = 128
NT = 16
EC = E // NT
NPAD = 10240
SL = NPAD // NT
LANES = 16


def _scores_tc(x_ref, w1_ref, wp_ref, bp_ref, sd_ref, ss_ref):
    w1 = w1_ref[...]
    wp = wp_ref[...]
    bp = bp_ref[0, 0]
    v1 = jax.lax.dot_general(wp[:, :D], w1, (((1,), (0,)), ((), ())),
                             preferred_element_type=jnp.float32)
    v2 = jax.lax.dot_general(wp[:, D:], w1, (((1,), (0,)), ((), ())),
                             preferred_element_type=jnp.float32)
    for k in range(2):
        xk = x_ref[k]
        sd = jax.lax.dot_general(v1, xk, (((1,), (1,)), ((), ())),
                                 preferred_element_type=jnp.float32)
        ss = jax.lax.dot_general(v2, xk, (((1,), (1,)), ((), ())),
                                 preferred_element_type=jnp.float32)
        sd_ref[pl.ds(k, 1), pl.ds(0, N)] = sd + bp
        ss_ref[pl.ds(k, 1), pl.ds(0, N)] = ss


def _out_tc(x_ref, w1_ref, a_ref, b_ref, o_ref):
    w1 = w1_ref[...]
    b = b_ref[...]
    for k in range(2):
        xk = x_ref[k]
        h = jax.lax.dot_general(xk, w1, (((1,), (1,)), ((), ())),
                                preferred_element_type=jnp.float32)
        o_ref[k] = h * a_ref[k] + b


def _edge_sc(sd_hbm, ss_hbm, src_hbm, dst_hbm, out_hbm,
             tbl_sd, tbl_ss, e_src, e_dst, acc_d, acc_s, red_d, red_s, a_sl,
             sh_pd, sh_ps):
    c = lax.axis_index("c")
    t = lax.axis_index("s")

    pltpu.sync_copy(sd_hbm.at[c], tbl_sd)
    pltpu.sync_copy(ss_hbm.at[c], tbl_ss)
    pltpu.sync_copy(src_hbm.at[c, t], e_src)
    pltpu.sync_copy(dst_hbm.at[c, t], e_dst)

    zeros = jnp.zeros((LANES,), jnp.float32)

    def zbody(i, carry):
        acc_d[pl.ds(i * LANES, LANES)] = zeros
        acc_s[pl.ds(i * LANES, LANES)] = zeros
        return carry

    lax.fori_loop(0, NPAD // LANES, zbody, 0)

    def ebody(i, carry):
        base = i * LANES
        d = e_dst[pl.ds(base, LANES)]
        s = e_src[pl.ds(base, LANES)]
        sc = plsc.load_gather(tbl_sd, [d]) + plsc.load_gather(tbl_ss, [s])
        lr = jnp.where(sc >= 0, sc, 0.2 * sc)
        num = jnp.exp(lr)
        plsc.addupdate_scatter(acc_d, [d], num)
        plsc.addupdate_scatter(acc_s, [s], num)
        return carry

    lax.fori_loop(0, EC // LANES, ebody, 0)

    pltpu.sync_copy(acc_d, sh_pd.at[t])
    pltpu.sync_copy(acc_s, sh_ps.at[t])
    plsc.subcore_barrier()

    for r in range(NT):
        pltpu.sync_copy(sh_pd.at[r, pl.ds(t * SL, SL)], red_d.at[r])
        pltpu.sync_copy(sh_ps.at[r, pl.ds(t * SL, SL)], red_s.at[r])

    def rbody(j, carry):
        jb = j * LANES
        den = red_d[0, pl.ds(jb, LANES)]
        sm = red_s[0, pl.ds(jb, LANES)]
        for r in range(1, NT):
            den = den + red_d[r, pl.ds(jb, LANES)]
            sm = sm + red_s[r, pl.ds(jb, LANES)]
        a_sl[pl.ds(jb, LANES)] = sm / (den + 1e-16)
        return carry

    lax.fori_loop(0, SL // LANES, rbody, 0)
    pltpu.sync_copy(a_sl, out_hbm.at[c, t])


@functools.cache
def _edge_kernel():
    return functools.partial(
        pl.kernel,
        mesh=plsc.VectorSubcoreMesh(core_axis_name="c", subcore_axis_name="s"),
        out_type=jax.ShapeDtypeStruct((2, NT, SL), jnp.float32),
        compiler_params=pltpu.CompilerParams(needs_layout_passes=False),
        scratch_types=[
            pltpu.VMEM((NPAD,), jnp.float32),
            pltpu.VMEM((NPAD,), jnp.float32),
            pltpu.VMEM((EC,), jnp.int32),
            pltpu.VMEM((EC,), jnp.int32),
            pltpu.VMEM((NPAD,), jnp.float32),
            pltpu.VMEM((NPAD,), jnp.float32),
            pltpu.VMEM((NT, SL), jnp.float32),
            pltpu.VMEM((NT, SL), jnp.float32),
            pltpu.VMEM((SL,), jnp.float32),
            pltpu.VMEM_SHARED((NT, NPAD), jnp.float32),
            pltpu.VMEM_SHARED((NT, NPAD), jnp.float32),
        ],
    )(_edge_sc)


@jax.jit
def kernel(x, edge_index, rank_mapping, W1, Wp, bp, bias):
    del rank_mapping
    src3 = edge_index[:, 0, :].reshape(2, NT, EC)
    dst3 = edge_index[:, 1, :].reshape(2, NT, EC)

    sd, ss = pl.pallas_call(
        _scores_tc,
        out_shape=[
            jax.ShapeDtypeStruct((2, NPAD), jnp.float32),
            jax.ShapeDtypeStruct((2, NPAD), jnp.float32),
        ],
        in_specs=[
            pl.BlockSpec(memory_space=pltpu.VMEM),
            pl.BlockSpec(memory_space=pltpu.VMEM),
            pl.BlockSpec(memory_space=pltpu.VMEM),
            pl.BlockSpec(memory_space=pltpu.SMEM),
        ],
    )(x, W1, Wp, bp.reshape(1, 1))

    a = _edge_kernel()(sd, ss, src3, dst3)
    a_col = a.reshape(2, NPAD)[:, :N, None]

    out = pl.pallas_call(
        _out_tc,
        out_shape=jax.ShapeDtypeStruct((2, N, D), jnp.float32),
    )(x, W1, a_col, bias.reshape(1, D))
    return out


# speedup vs baseline: 330.6256x; 1.2157x over previous
import functools

import jax
import jax.numpy as jnp
from jax import lax
from jax.experimental import pallas as pl
from jax.experimental.pallas import tpu as pltpu
from jax.experimental.pallas import tpu_sc as plsc

N = 10000
E = 320000
D = 128
NT = 16
EC = E // NT
NPAD = 10240
SL = NPAD // NT
LANES = 16


def _scores_tc(x_ref, w1_ref, wp_ref, bp_ref, sd_ref, ss_ref):
    w1 = w1_ref[...]
    wp = wp_ref[...]
    bp = bp_ref[0, 0]
    v1 = jax.lax.dot_general(wp[:, :D], w1, (((1,), (0,)), ((), ())),
                             preferred_element_type=jnp.float32)
    v2 = jax.lax.dot_general(wp[:, D:], w1, (((1,), (0,)), ((), ())),
                             preferred_element_type=jnp.float32)
    for k in range(2):
        xk = x_ref[k]
        sd = jax.lax.dot_general(v1, xk, (((1,), (1,)), ((), ())),
                                 preferred_element_type=jnp.float32)
        ss = jax.lax.dot_general(v2, xk, (((1,), (1,)), ((), ())),
                                 preferred_element_type=jnp.float32)
        sd_ref[pl.ds(k, 1), pl.ds(0, N)] = sd + bp
        ss_ref[pl.ds(k, 1), pl.ds(0, N)] = ss


def _out_tc(x_ref, w1_ref, a_ref, b_ref, o_ref):
    w1 = w1_ref[...]
    b = b_ref[...]
    for k in range(2):
        xk = x_ref[k]
        h = jax.lax.dot_general(xk, w1, (((1,), (1,)), ((), ())),
                                preferred_element_type=jnp.float32)
        o_ref[k] = h * a_ref[k] + b


_UNROLL = 10


def _edge_sc(sd_hbm, ss_hbm, ei_hbm, out_hbm,
             tbl_sd, tbl_ss, e_src, e_dst, acc_d, acc_s, red_d, red_s, a_sl,
             sh_pd, sh_ps, sem):
    c = lax.axis_index("c")
    t = lax.axis_index("s")

    pltpu.sync_copy(sd_hbm.at[c], tbl_sd)
    pltpu.sync_copy(ss_hbm.at[c], tbl_ss)
    ebase = c * (2 * E) + t * EC
    pltpu.sync_copy(ei_hbm.at[pl.ds(ebase, EC)], e_src)
    pltpu.sync_copy(ei_hbm.at[pl.ds(ebase + E, EC)], e_dst)

    zeros = jnp.zeros((LANES,), jnp.float32)

    def zbody(i, carry):
        for u in range(_UNROLL):
            base = (i * _UNROLL + u) * LANES
            acc_d[pl.ds(base, LANES)] = zeros
            acc_s[pl.ds(base, LANES)] = zeros
        return carry

    lax.fori_loop(0, NPAD // LANES // _UNROLL, zbody, 0)

    def ebody(i, carry):
        for u in range(_UNROLL):
            base = (i * _UNROLL + u) * LANES
            d = e_dst[pl.ds(base, LANES)]
            s = e_src[pl.ds(base, LANES)]
            sc = plsc.load_gather(tbl_sd, [d]) + plsc.load_gather(tbl_ss, [s])
            lr = jnp.where(sc >= 0, sc, 0.2 * sc)
            num = jnp.exp(lr)
            plsc.addupdate_scatter(acc_d, [d], num)
            plsc.addupdate_scatter(acc_s, [s], num)
        return carry

    lax.fori_loop(0, EC // LANES // _UNROLL, ebody, 0)

    pltpu.sync_copy(acc_d, sh_pd.at[t])
    pltpu.sync_copy(acc_s, sh_ps.at[t])
    plsc.subcore_barrier()

    copies = []
    for r in range(NT):
        copies.append(pltpu.async_copy(
            sh_pd.at[r, pl.ds(t * SL, SL)], red_d.at[r], sem))
        copies.append(pltpu.async_copy(
            sh_ps.at[r, pl.ds(t * SL, SL)], red_s.at[r], sem))
    for cp in copies:
        cp.wait()

    def rbody(j, carry):
        jb = j * LANES
        den = red_d[0, pl.ds(jb, LANES)]
        sm = red_s[0, pl.ds(jb, LANES)]
        for r in range(1, NT):
            den = den + red_d[r, pl.ds(jb, LANES)]
            sm = sm + red_s[r, pl.ds(jb, LANES)]
        a_sl[pl.ds(jb, LANES)] = sm / (den + 1e-16)
        return carry

    lax.fori_loop(0, SL // LANES, rbody, 0)
    pltpu.sync_copy(a_sl, out_hbm.at[c, t])


@functools.cache
def _edge_kernel():
    return functools.partial(
        pl.kernel,
        mesh=plsc.VectorSubcoreMesh(core_axis_name="c", subcore_axis_name="s"),
        out_type=jax.ShapeDtypeStruct((2, NT, SL), jnp.float32),
        compiler_params=pltpu.CompilerParams(needs_layout_passes=False),
        scratch_types=[
            pltpu.VMEM((NPAD,), jnp.float32),
            pltpu.VMEM((NPAD,), jnp.float32),
            pltpu.VMEM((EC,), jnp.int32),
            pltpu.VMEM((EC,), jnp.int32),
            pltpu.VMEM((NPAD,), jnp.float32),
            pltpu.VMEM((NPAD,), jnp.float32),
            pltpu.VMEM((NT, SL), jnp.float32),
            pltpu.VMEM((NT, SL), jnp.float32),
            pltpu.VMEM((SL,), jnp.float32),
            pltpu.VMEM_SHARED((NT, NPAD), jnp.float32),
            pltpu.VMEM_SHARED((NT, NPAD), jnp.float32),
            pltpu.SemaphoreType.DMA,
        ],
    )(_edge_sc)


@jax.jit
def kernel(x, edge_index, rank_mapping, W1, Wp, bp, bias):
    del rank_mapping

    sd, ss = pl.pallas_call(
        _scores_tc,
        out_shape=[
            jax.ShapeDtypeStruct((2, NPAD), jnp.float32),
            jax.ShapeDtypeStruct((2, NPAD), jnp.float32),
        ],
        in_specs=[
            pl.BlockSpec(memory_space=pltpu.VMEM),
            pl.BlockSpec(memory_space=pltpu.VMEM),
            pl.BlockSpec(memory_space=pltpu.VMEM),
            pl.BlockSpec(memory_space=pltpu.SMEM),
        ],
    )(x, W1, Wp, bp.reshape(1, 1))

    a = _edge_kernel()(sd, ss, edge_index.reshape(-1))
    a_col = a.reshape(2, NPAD)[:, :N, None]

    out = pl.pallas_call(
        _out_tc,
        out_shape=jax.ShapeDtypeStruct((2, N, D), jnp.float32),
    )(x, W1, a_col, bias.reshape(1, D))
    return out


# speedup vs baseline: 651.2931x; 1.9699x over previous
import functools

import jax
import jax.numpy as jnp
from jax import lax
from jax.experimental import pallas as pl
from jax.experimental.pallas import tpu as pltpu
from jax.experimental.pallas import tpu_sc as plsc

N = 10000
E = 320000
D = 128
NT = 16
ECA = 19968
ECB = E - 15 * ECA
ECH = 10240
NPAD = 10240
SL = NPAD // NT
LANES = 16


def _scores_tc(x_ref, w1_ref, wp_ref, bp_ref, sd_ref, ss_ref, h_ref):
    w1 = w1_ref[...]
    wp = wp_ref[...]
    bp = bp_ref[0, 0]
    xk = x_ref[0]
    h = jax.lax.dot_general(xk, w1, (((1,), (1,)), ((), ())),
                            preferred_element_type=jnp.float32)
    h_ref[0] = h.astype(jnp.bfloat16)
    sd = jax.lax.dot_general(wp[:, :D], h, (((1,), (1,)), ((), ())),
                             preferred_element_type=jnp.float32)
    ss = jax.lax.dot_general(wp[:, D:], h, (((1,), (1,)), ((), ())),
                             preferred_element_type=jnp.float32)
    sd_ref[0, pl.ds(0, 1), pl.ds(0, N)] = sd + bp
    ss_ref[0, pl.ds(0, 1), pl.ds(0, N)] = ss


def _out_tc(h_ref, a_ref, b_ref, o_ref):
    b = b_ref[...]
    hk = h_ref[0].astype(jnp.float32)
    ak = a_ref[0, 0, pl.ds(0, N)]
    o_ref[0] = hk * ak[:, None] + b


_UNROLL = 4


def _edge_sc(sd_hbm, ss_hbm, ei_hbm, out_hbm,
             tbl_sd, tbl_ss, e_buf, acc_d, acc_s, red_d, red_s, a_sl,
             sh_pd, sh_ps, sem):
    c = lax.axis_index("c")
    t = lax.axis_index("s")

    in_copies = [
        pltpu.async_copy(sd_hbm.at[c, 0], tbl_sd, sem),
        pltpu.async_copy(ss_hbm.at[c, 0], tbl_ss, sem),
        pltpu.async_copy(ei_hbm.at[c, :, pl.ds(t * ECA, ECH)],
                         e_buf.at[:, pl.ds(0, ECH)], sem),
    ]
    cp2 = pltpu.async_copy(ei_hbm.at[c, :, pl.ds(t * ECA + ECH, ECB - ECH)],
                           e_buf.at[:, pl.ds(ECH, ECB - ECH)], sem)

    zeros = jnp.zeros((LANES,), jnp.float32)

    @plsc.parallel_loop(0, NPAD // LANES, unroll=_UNROLL)
    def _(i):
        acc_d[pl.ds(i * LANES, LANES)] = zeros
        acc_s[pl.ds(i * LANES, LANES)] = zeros

    for cp in in_copies:
        cp.wait()

    def _edge_vec(i):
        base = i * LANES
        s = e_buf[0, pl.ds(base, LANES)]
        d = e_buf[1, pl.ds(base, LANES)]
        sc = plsc.load_gather(tbl_sd, [d]) + plsc.load_gather(tbl_ss, [s])
        lr = jnp.where(sc >= 0, sc, 0.2 * sc)
        num = jnp.exp(lr)
        plsc.addupdate_scatter(acc_d, [d], num)
        plsc.addupdate_scatter(acc_s, [s], num)

    plsc.parallel_loop(0, ECH // LANES, unroll=_UNROLL)(_edge_vec)
    cp2.wait()
    plsc.parallel_loop(ECH // LANES, ECA // LANES, unroll=_UNROLL)(_edge_vec)

    @pl.when(t == NT - 1)
    def _():
        plsc.parallel_loop(ECA // LANES, ECB // LANES, unroll=_UNROLL)(_edge_vec)

    pltpu.sync_copy(acc_d, sh_pd.at[t])
    pltpu.sync_copy(acc_s, sh_ps.at[t])
    plsc.subcore_barrier()

    copies = []
    for r in range(NT):
        copies.append(pltpu.async_copy(
            sh_pd.at[r, pl.ds(t * SL, SL)], red_d.at[r], sem))
        copies.append(pltpu.async_copy(
            sh_ps.at[r, pl.ds(t * SL, SL)], red_s.at[r], sem))
    for cp in copies:
        cp.wait()

    @plsc.parallel_loop(0, SL // LANES, unroll=2)
    def _(j):
        jb = j * LANES
        den = red_d[0, pl.ds(jb, LANES)]
        sm = red_s[0, pl.ds(jb, LANES)]
        for r in range(1, NT):
            den = den + red_d[r, pl.ds(jb, LANES)]
            sm = sm + red_s[r, pl.ds(jb, LANES)]
        a_sl[pl.ds(jb, LANES)] = sm / (den + 1e-16)
    pltpu.sync_copy(a_sl, out_hbm.at[c, 0, pl.ds(t * SL, SL)])


@functools.cache
def _edge_kernel():
    return functools.partial(
        pl.kernel,
        mesh=plsc.VectorSubcoreMesh(core_axis_name="c", subcore_axis_name="s"),
        out_type=jax.ShapeDtypeStruct((2, 1, NPAD), jnp.float32),
        compiler_params=pltpu.CompilerParams(needs_layout_passes=False),
        scratch_types=[
            pltpu.VMEM((NPAD,), jnp.float32),
            pltpu.VMEM((NPAD,), jnp.float32),
            pltpu.VMEM((2, ECB), jnp.int32),
            pltpu.VMEM((NPAD,), jnp.float32),
            pltpu.VMEM((NPAD,), jnp.float32),
            pltpu.VMEM((NT, SL), jnp.float32),
            pltpu.VMEM((NT, SL), jnp.float32),
            pltpu.VMEM((SL,), jnp.float32),
            pltpu.VMEM_SHARED((NT, NPAD), jnp.float32),
            pltpu.VMEM_SHARED((NT, NPAD), jnp.float32),
            pltpu.SemaphoreType.DMA,
        ],
    )(_edge_sc)


@jax.jit
def kernel(x, edge_index, rank_mapping, W1, Wp, bp, bias):
    del rank_mapping

    sd, ss, h16 = pl.pallas_call(
        _scores_tc,
        grid=(2,),
        out_shape=[
            jax.ShapeDtypeStruct((2, 1, NPAD), jnp.float32),
            jax.ShapeDtypeStruct((2, 1, NPAD), jnp.float32),
            jax.ShapeDtypeStruct((2, N, D), jnp.bfloat16),
        ],
        in_specs=[
            pl.BlockSpec((1, N, D), lambda k: (k, 0, 0)),
            pl.BlockSpec((D, D), lambda k: (0, 0)),
            pl.BlockSpec((1, 2 * D), lambda k: (0, 0)),
            pl.BlockSpec(memory_space=pltpu.SMEM),
        ],
        out_specs=[
            pl.BlockSpec((1, 1, NPAD), lambda k: (k, 0, 0)),
            pl.BlockSpec((1, 1, NPAD), lambda k: (k, 0, 0)),
            pl.BlockSpec((1, N, D), lambda k: (k, 0, 0)),
        ],
    )(x, W1, Wp, bp.reshape(1, 1))

    a = _edge_kernel()(sd, ss, edge_index)

    out = pl.pallas_call(
        _out_tc,
        grid=(2,),
        out_shape=jax.ShapeDtypeStruct((2, N, D), jnp.float32),
        in_specs=[
            pl.BlockSpec((1, N, D), lambda k: (k, 0, 0)),
            pl.BlockSpec((1, 1, NPAD), lambda k: (k, 0, 0)),
            pl.BlockSpec((1, D), lambda k: (0, 0)),
        ],
        out_specs=pl.BlockSpec((1, N, D), lambda k: (k, 0, 0)),
    )(h16, a, bias.reshape(1, D))
    return out
